# baseline (speedup 1.0000x reference)
import functools

import jax
import jax.numpy as jnp
from jax import lax
from jax.experimental import pallas as pl
from jax.experimental.pallas import tpu as pltpu
from jax.experimental.pallas import tpu_sc as plsc

EMBED_DIM = 768
BATCH = 4
SEQ = 2048

NUM_CORES = 2
NUM_SUBCORES = 16
NUM_WORKERS = NUM_CORES * NUM_SUBCORES
SLICE = SEQ // NUM_WORKERS
CHUNK = 16
NBUF = 4
NCHUNKS = BATCH * SLICE // CHUNK
VECS = EMBED_DIM // 16


def _emb_body(x_hbm, pos_hbm, table_hbm, out_hbm, idx_all, pos_v,
              rb0, rb1, rb2, rb3, dummy, g0, g1, g2, g3, s0_, s1_, s2_, s3_,
              psem, isem):
    rows = (rb0, rb1, rb2, rb3)
    gsems = (g0, g1, g2, g3)
    ssems = (s0_, s1_, s2_, s3_)
    wid = lax.axis_index("s") * NUM_CORES + lax.axis_index("c")
    seq0 = wid * SLICE

    pos_cp = pltpu.async_copy(pos_hbm.at[pl.ds(seq0, SLICE), :], pos_v, psem)
    idx_cps = [
        pltpu.async_copy(x_hbm.at[b, pl.ds(seq0, SLICE)],
                         idx_all.at[b], isem)
        for b in range(BATCH)
    ]
    for cp in idx_cps:
        cp.wait()

    def start_gather(t):
        b, q = t // 4, t % 4
        p = t % NBUF
        return pltpu.async_copy(
            table_hbm.at[idx_all.at[b, pl.ds(q * CHUNK, CHUNK)]],
            rows[p], gsems[p])

    def start_store(t):
        b, q = t // 4, t % 4
        p = t % NBUF
        base = b * SEQ + seq0 + q * CHUNK
        return pltpu.async_copy(rows[p], out_hbm.at[pl.ds(base, CHUNK), :],
                                ssems[p])

    gh = [None] * NCHUNKS
    sh = [None] * NCHUNKS
    gh[0] = start_gather(0)
    gh[1] = start_gather(1)
    pos_cp.wait()

    for t in range(NCHUNKS):
        if t + 2 < NCHUNKS:
            if t - 2 >= 0:
                sh[t - 2].wait()
            gh[t + 2] = start_gather(t + 2)
        gh[t].wait()

        p = t % NBUF
        prow = (t % 4) * CHUNK
        rbuf = rows[p]

        def add_row(r, carry):
            for c in range(VECS):
                sl = pl.ds(c * 16, 16)
                plsc.addupdate(dummy.at[r, sl], pos_v[prow + r, sl])
            return carry

        lax.fori_loop(0, CHUNK, add_row, 0)
        sh[t] = start_store(t)

    for t in range(NCHUNKS - 4, NCHUNKS):
        sh[t].wait()


@jax.jit
def _emb(x2d, pos2d, table):
    mesh = plsc.VectorSubcoreMesh(core_axis_name="c", subcore_axis_name="s")
    run = functools.partial(
        pl.kernel,
        out_type=jax.ShapeDtypeStruct((BATCH * SEQ, EMBED_DIM), jnp.float32),
        mesh=mesh,
        scratch_types=[
            pltpu.VMEM((BATCH, SLICE), jnp.int32),
            pltpu.VMEM((SLICE, EMBED_DIM), jnp.float32),
        ] + [pltpu.VMEM((CHUNK, EMBED_DIM), jnp.float32)] * (NBUF + 1)
        + [pltpu.SemaphoreType.DMA] * (2 * NBUF + 2),
    )(_emb_body)
    return run(x2d, pos2d, table)


def kernel(x, token_table, position_embedding):
    x2d = x.astype(jnp.int32)
    pos2d = position_embedding[0, : x.shape[1], :]
    out = _emb(x2d, pos2d, token_table)
    return out.reshape(x.shape[0], x.shape[1], EMBED_DIM)


# speedup vs baseline: 1.1711x; 1.1711x over previous
import functools

import jax
import jax.numpy as jnp
from jax import lax
from jax.experimental import pallas as pl
from jax.experimental.pallas import tpu as pltpu
from jax.experimental.pallas import tpu_sc as plsc

EMBED_DIM = 768
BATCH = 4
SEQ = 2048

NUM_CORES = 2
NUM_SUBCORES = 16
NUM_WORKERS = NUM_CORES * NUM_SUBCORES
SLICE = SEQ // NUM_WORKERS
CHUNK = 16
QSTEPS = SLICE // CHUNK
VECS = EMBED_DIM // 16


def _emb_body(x_hbm, pos_hbm, table_hbm, out_hbm, *scr):
    idx_all = scr[0]
    pb = scr[1:3]
    rb = scr[3:11]
    isem = scr[11]
    psem = scr[12:14]
    gsem = scr[14:22]
    ssem = scr[22:30]

    wid = lax.axis_index("s") * NUM_CORES + lax.axis_index("c")
    seq0 = wid * SLICE

    idx_cps = [
        pltpu.async_copy(x_hbm.at[b, pl.ds(seq0, SLICE)], idx_all.at[b], isem)
        for b in range(BATCH)
    ]
    for cp in idx_cps:
        cp.wait()

    def issue_q(q):
        p = q % 2
        pos_h = pltpu.async_copy(
            pos_hbm.at[pl.ds(seq0 + q * CHUNK, CHUNK), :], pb[p], psem[p])
        g_h = [
            pltpu.async_copy(
                table_hbm.at[idx_all.at[b, pl.ds(q * CHUNK, CHUNK)]],
                rb[p * 4 + b], gsem[p * 4 + b])
            for b in range(BATCH)
        ]
        return pos_h, g_h

    def issue_stores(q):
        p = q % 2
        return [
            pltpu.async_copy(
                rb[p * 4 + b],
                out_hbm.at[pl.ds(b * SEQ + seq0 + q * CHUNK, CHUNK), :],
                ssem[p * 4 + b])
            for b in range(BATCH)
        ]

    hs = {0: issue_q(0), 1: issue_q(1)}
    st = {}
    for q in range(QSTEPS):
        p = q % 2
        pos_h, g_h = hs[q]
        pos_h.wait()
        for h in g_h:
            h.wait()

        pbuf = pb[p]
        r0, r1, r2, r3 = rb[p * 4:p * 4 + 4]

        def add_row(r, carry):
            for c in range(VECS):
                sl = pl.ds(c * 16, 16)
                v = pbuf[r, sl]
                plsc.addupdate(r0.at[r, sl], v)
                plsc.addupdate(r1.at[r, sl], v)
                plsc.addupdate(r2.at[r, sl], v)
                plsc.addupdate(r3.at[r, sl], v)
            return carry

        lax.fori_loop(0, CHUNK, add_row, 0)
        st[q] = issue_stores(q)
        if q >= 1 and q + 1 < QSTEPS:
            for h in st[q - 1]:
                h.wait()
            hs[q + 1] = issue_q(q + 1)

    for q in (QSTEPS - 2, QSTEPS - 1):
        for h in st[q]:
            h.wait()


@jax.jit
def _emb(x2d, pos2d, table):
    mesh = plsc.VectorSubcoreMesh(core_axis_name="c", subcore_axis_name="s")
    run = functools.partial(
        pl.kernel,
        out_type=jax.ShapeDtypeStruct((BATCH * SEQ, EMBED_DIM), jnp.float32),
        mesh=mesh,
        scratch_types=[
            pltpu.VMEM((BATCH, SLICE), jnp.int32),
        ] + [pltpu.VMEM((CHUNK, EMBED_DIM), jnp.float32)] * 2
        + [pltpu.VMEM((CHUNK, EMBED_DIM), jnp.float32)] * 8
        + [pltpu.SemaphoreType.DMA] * 19,
    )(_emb_body)
    return run(x2d, pos2d, table)


def kernel(x, token_table, position_embedding):
    x2d = x.astype(jnp.int32)
    pos2d = position_embedding[0, : x.shape[1], :]
    out = _emb(x2d, pos2d, token_table)
    return out.reshape(x.shape[0], x.shape[1], EMBED_DIM)


# speedup vs baseline: 1.1886x; 1.0150x over previous
import functools

import jax
import jax.numpy as jnp
from jax import lax
from jax.experimental import pallas as pl
from jax.experimental.pallas import tpu as pltpu
from jax.experimental.pallas import tpu_sc as plsc

EMBED_DIM = 768
BATCH = 4
SEQ = 2048

NUM_CORES = 2
NUM_SUBCORES = 16
NUM_WORKERS = NUM_CORES * NUM_SUBCORES
SLICE = SEQ // NUM_WORKERS
CHUNK = 16
QSTEPS = SLICE // CHUNK
VECS = EMBED_DIM // 16


def _emb_body(x_hbm, pos_hbm, table_hbm, out_hbm, *scr):
    idx_all = scr[0]
    pb = scr[1:3]
    rb = scr[3:11]
    isem = scr[11]
    psem = scr[12:14]
    gsem = scr[14:16]
    ssem = scr[16:18]

    wid = lax.axis_index("s") * NUM_CORES + lax.axis_index("c")
    seq0 = wid * SLICE

    idx_cps = [
        pltpu.async_copy(x_hbm.at[b, pl.ds(seq0, SLICE)], idx_all.at[b], isem)
        for b in range(BATCH)
    ]
    for cp in idx_cps:
        cp.wait()

    def issue_q(q):
        p = q % 2
        pos_h = pltpu.async_copy(
            pos_hbm.at[pl.ds(seq0 + q * CHUNK, CHUNK), :], pb[p], psem[p])
        g_h = [
            pltpu.async_copy(
                table_hbm.at[idx_all.at[b, pl.ds(q * CHUNK, CHUNK)]],
                rb[p * 4 + b], gsem[p])
            for b in range(BATCH)
        ]
        return pos_h, g_h

    def issue_stores(q):
        p = q % 2
        return [
            pltpu.async_copy(
                rb[p * 4 + b],
                out_hbm.at[pl.ds(b * SEQ + seq0 + q * CHUNK, CHUNK), :],
                ssem[p])
            for b in range(BATCH)
        ]

    hs = {0: issue_q(0), 1: issue_q(1)}
    st = {}
    for q in range(QSTEPS):
        p = q % 2
        pos_h, g_h = hs[q]
        pos_h.wait()
        for h in g_h:
            h.wait()

        pbuf = pb[p]
        r0, r1, r2, r3 = rb[p * 4:p * 4 + 4]

        def add_row(r, carry):
            for c in range(VECS):
                sl = pl.ds(c * 16, 16)
                v = pbuf[r, sl]
                plsc.addupdate(r0.at[r, sl], v)
                plsc.addupdate(r1.at[r, sl], v)
                plsc.addupdate(r2.at[r, sl], v)
                plsc.addupdate(r3.at[r, sl], v)
            return carry

        lax.fori_loop(0, CHUNK, add_row, 0)
        st[q] = issue_stores(q)
        if q >= 1 and q + 1 < QSTEPS:
            for h in st[q - 1]:
                h.wait()
            hs[q + 1] = issue_q(q + 1)

    for q in (QSTEPS - 2, QSTEPS - 1):
        for h in st[q]:
            h.wait()


@jax.jit
def _emb(x2d, pos2d, table):
    mesh = plsc.VectorSubcoreMesh(core_axis_name="c", subcore_axis_name="s")
    run = functools.partial(
        pl.kernel,
        out_type=jax.ShapeDtypeStruct((BATCH * SEQ, EMBED_DIM), jnp.float32),
        mesh=mesh,
        scratch_types=[
            pltpu.VMEM((BATCH, SLICE), jnp.int32),
        ] + [pltpu.VMEM((CHUNK, EMBED_DIM), jnp.float32)] * 2
        + [pltpu.VMEM((CHUNK, EMBED_DIM), jnp.float32)] * 8
        + [pltpu.SemaphoreType.DMA] * 7,
    )(_emb_body)
    return run(x2d, pos2d, table)


def kernel(x, token_table, position_embedding):
    x2d = x.astype(jnp.int32)
    pos2d = position_embedding[0, : x.shape[1], :]
    out = _emb(x2d, pos2d, token_table)
    return out.reshape(x.shape[0], x.shape[1], EMBED_DIM)
